# baseline (speedup 1.0000x reference)
import functools

import jax
import jax.numpy as jnp
from jax import lax
from jax.experimental import pallas as pl
from jax.experimental.pallas import tpu as pltpu
from jax.experimental.pallas import tpu_sc as plsc

N_TILES = 32
N_SUB = 16
K = 128
LANES = 16
NBUF = 2


def _mesh():
    return plsc.VectorSubcoreMesh(core_axis_name="c", subcore_axis_name="s")


def _sc_params():
    return pltpu.CompilerParams(needs_layout_passes=False)


def _make_deg_kernel(npad, ch):
    sl = npad // N_SUB
    assert ch % 8 == 0

    @functools.partial(
        pl.kernel,
        out_type=jax.ShapeDtypeStruct((2, npad), jnp.float32),
        mesh=_mesh(),
        compiler_params=_sc_params(),
        scratch_types=[
            pltpu.VMEM_SHARED((npad,), jnp.float32),
            pltpu.VMEM((ch, K), jnp.int32),
            pltpu.VMEM((K,), jnp.float32),
            pltpu.SemaphoreType.DMA,
        ],
    )
    def deg_kernel(dst_hbm, zvec_hbm, out_hbm, deg_sp, dst_v, ones_v, sem):
        c = lax.axis_index("c")
        s = lax.axis_index("s")
        w = c * N_SUB + s
        pltpu.sync_copy(zvec_hbm.at[pl.ds(s * sl, sl)], deg_sp.at[pl.ds(s * sl, sl)])
        pltpu.sync_copy(dst_hbm.at[w], dst_v)
        for k in range(K // LANES):
            ones_v[pl.ds(k * LANES, LANES)] = jnp.full((LANES,), 1.0, jnp.float32)
        plsc.subcore_barrier()

        def body(i, carry):
            pltpu.sync_copy(ones_v, deg_sp.at[dst_v.at[i]], add=True)
            return carry

        lax.fori_loop(0, ch, body, 0)
        plsc.subcore_barrier()
        pltpu.sync_copy(deg_sp.at[pl.ds(s * sl, sl)], out_hbm.at[c, pl.ds(s * sl, sl)])

    return deg_kernel


def _make_row_scatter_kernel(npad, dh, ch, k3, gr):
    sl = npad // N_SUB
    kw = k3 // 2
    ngrp = ch // gr
    assert ngrp % 2 == 0 and gr % 2 == 0

    @functools.partial(
        pl.kernel,
        out_type=jax.ShapeDtypeStruct((2, npad, dh), jnp.float32),
        mesh=_mesh(),
        compiler_params=_sc_params(),
        scratch_types=[
            pltpu.VMEM_SHARED((npad, dh), jnp.float32),
            pltpu.VMEM((2, gr, kw), jnp.int32),
            pltpu.VMEM((2, gr, kw), jnp.int32),
            pltpu.VMEM((NBUF, k3), jnp.int32),
            pltpu.VMEM((NBUF, k3), jnp.int32),
        ]
        + [pltpu.VMEM((k3, dh), jnp.float32) for _ in range(NBUF)]
        + [pltpu.SemaphoreType.DMA for _ in range(NBUF)]
        + [pltpu.SemaphoreType.DMA],
    )
    def scat_kernel(g_hbm, src_hbm, dst_hbm, zrows_hbm, out_hbm,
                    acc_sp, gsrc, gdst, sbufs, dbufs, *rest):
        bufs = rest[:NBUF]
        gsem = rest[NBUF:2 * NBUF]
        isem = rest[2 * NBUF]
        c = lax.axis_index("c")
        s = lax.axis_index("s")
        w = c * N_SUB + s
        pltpu.sync_copy(zrows_hbm.at[pl.ds(s * sl, sl)], acc_sp.at[pl.ds(s * sl, sl)])
        plsc.subcore_barrier()

        def fire_idx(jg, t):
            pltpu.async_copy(src_hbm.at[w, pl.ds(jg * gr, gr)], gsrc.at[t], isem)
            pltpu.async_copy(dst_hbm.at[w, pl.ds(jg * gr, gr)], gdst.at[t], isem)

        def wait_idx():
            pltpu.make_async_copy(src_hbm.at[w, pl.ds(0, gr)], gsrc.at[0], isem).wait()
            pltpu.make_async_copy(dst_hbm.at[w, pl.ds(0, gr)], gdst.at[0], isem).wait()

        def unpack_idx(t, i, b):
            for q in range(kw // LANES):
                ws = gsrc[t, i, pl.ds(q * LANES, LANES)]
                sbufs[b, pl.ds(q * LANES, LANES)] = ws & 0xFFFF
                sbufs[b, pl.ds(kw + q * LANES, LANES)] = lax.shift_right_logical(ws, 16)
                wd = gdst[t, i, pl.ds(q * LANES, LANES)]
                dbufs[b, pl.ds(q * LANES, LANES)] = wd & 0xFFFF
                dbufs[b, pl.ds(kw + q * LANES, LANES)] = lax.shift_right_logical(wd, 16)

        def process_group(t, jg):
            for b in range(NBUF):
                unpack_idx(t, b, b)
                pltpu.async_copy(g_hbm.at[sbufs.at[b]], bufs[b], gsem[b])

            def inner(p, carry):
                for b in range(NBUF):
                    i = p * NBUF + b
                    pltpu.make_async_copy(g_hbm.at[sbufs.at[b]], bufs[b], gsem[b]).wait()
                    pltpu.sync_copy(bufs[b], acc_sp.at[dbufs.at[b]], add=True)
                    unpack_idx(t, i + NBUF, b)
                    pltpu.async_copy(g_hbm.at[sbufs.at[b]], bufs[b], gsem[b])
                return carry

            lax.fori_loop(0, gr // NBUF - 1, inner, 0)
            for b in range(NBUF):
                pltpu.make_async_copy(g_hbm.at[sbufs.at[b]], bufs[b], gsem[b]).wait()
                pltpu.sync_copy(bufs[b], acc_sp.at[dbufs.at[b]], add=True)
            @pl.when(jg + 2 < ngrp)
            def _():
                fire_idx(jg + 2, t)

        pltpu.sync_copy(src_hbm.at[w, pl.ds(0, gr)], gsrc.at[0])
        pltpu.sync_copy(dst_hbm.at[w, pl.ds(0, gr)], gdst.at[0])
        fire_idx(1, 1)

        def outer(q, carry):
            jg0 = q * 2

            @pl.when(q > 0)
            def _():
                wait_idx()

            process_group(0, jg0)
            wait_idx()
            process_group(1, jg0 + 1)
            return carry

        lax.fori_loop(0, ngrp // 2, outer, 0)
        plsc.subcore_barrier()
        pltpu.sync_copy(acc_sp.at[pl.ds(s * sl, sl)], out_hbm.at[c, pl.ds(s * sl, sl)])

    return scat_kernel


def _make_scalar_kernel(npad, ch):
    sl = npad // N_SUB
    assert ch % 2 == 0

    @functools.partial(
        pl.kernel,
        out_type=jax.ShapeDtypeStruct((2, npad), jnp.float32),
        mesh=_mesh(),
        compiler_params=_sc_params(),
        scratch_types=[
            pltpu.VMEM_SHARED((npad,), jnp.float32),
            pltpu.VMEM((ch, K), jnp.int32),
            pltpu.VMEM((ch, K), jnp.int32),
            pltpu.VMEM((npad // 128, 128), jnp.float32),
            pltpu.VMEM((K,), jnp.float32),
            pltpu.VMEM((K,), jnp.float32),
            pltpu.VMEM((sl,), jnp.float32),
            pltpu.VMEM((sl,), jnp.float32),
            pltpu.VMEM((LANES,), jnp.float32),
            pltpu.SemaphoreType.DMA,
            pltpu.SemaphoreType.DMA,
        ],
    )
    def l2agg_kernel(z_hbm, dinv_hbm, b2_hbm, src_hbm, dst_hbm, zvec_hbm, out_hbm,
                     acc_sp, src_v, dst_v, z_v, upd_a, upd_b, acc_v, dinv_v, b2_v,
                     sem_a, sem_b):
        c = lax.axis_index("c")
        s = lax.axis_index("s")
        pltpu.sync_copy(zvec_hbm.at[pl.ds(s * sl, sl)], acc_sp.at[pl.ds(s * sl, sl)])
        pltpu.sync_copy(src_hbm.at[s], src_v)
        pltpu.sync_copy(dst_hbm.at[s], dst_v)
        pltpu.sync_copy(z_hbm, z_v)
        pltpu.sync_copy(dinv_hbm.at[pl.ds(s * sl, sl)], dinv_v)
        pltpu.sync_copy(b2_hbm, b2_v)
        plsc.subcore_barrier()

        def gather_chunk(i, upd_v):
            for k in range(K // LANES):
                s16 = src_v[i, pl.ds(k * LANES, LANES)]
                r16 = lax.shift_right_logical(s16, 7)
                c16 = lax.bitwise_and(s16, 127)
                upd_v[pl.ds(k * LANES, LANES)] = plsc.load_gather(z_v, [r16, c16])

        def body(i, carry):
            gather_chunk(i, upd_a)
            pltpu.sync_copy(upd_a, acc_sp.at[dst_v.at[i]], add=True)
            return carry

        lax.fori_loop(0, ch, body, 0)
        plsc.subcore_barrier()
        pltpu.sync_copy(acc_sp.at[pl.ds(s * sl, sl)], acc_v)
        b2 = b2_v[...]
        rows_per_sub = sl // 128
        for t in range(sl // LANES):
            a16 = acc_v[pl.ds(t * LANES, LANES)]
            z16 = z_v[s * rows_per_sub + t // 8, pl.ds((t % 8) * LANES, LANES)]
            d16 = dinv_v[pl.ds(t * LANES, LANES)]
            acc_v[pl.ds(t * LANES, LANES)] = b2 + d16 * (a16 + z16)
        pltpu.sync_copy(acc_v, out_hbm.at[c, pl.ds(s * sl, sl)])

    return l2agg_kernel


def _g_body(x_ref, w1_ref, degt_ref, g_ref, dinv_ref):
    d = lax.rsqrt(degt_ref[:, 0:1] + degt_ref[:, 1:2] + 1.0)
    h = jnp.dot(x_ref[...], w1_ref[...], preferred_element_type=jnp.float32)
    g_ref[...] = d * h
    dinv_ref[...] = d


def _l4_body(acc_ref, g_ref, dinv_ref, w2_ref, b1_ref, z_ref):
    d = dinv_ref[...]
    h1 = jnp.maximum(d * (acc_ref[0] + acc_ref[1] + g_ref[...]) + b1_ref[...], 0.0)
    z_ref[...] = d * jnp.dot(h1, w2_ref[...], preferred_element_type=jnp.float32)


def kernel(x, edge_index, W1, b1, W2, b2):
    n, d_in = x.shape
    d_hid = W1.shape[1]
    dh = d_hid // 2
    ei = edge_index.astype(jnp.int32)
    src, dst = ei[0], ei[1]
    e = src.shape[0]

    npad = ((n + 16 * 40 - 1) // (16 * 40)) * (16 * 40)
    ch = ((e + N_TILES * K - 1) // (N_TILES * K) + 7) // 8 * 8
    e_pad = N_TILES * ch * K
    n_extra = e_pad - e
    pad_idx = jnp.arange(n_extra, dtype=jnp.int32)
    src_p = jnp.concatenate([src, pad_idx % n])
    dst_p = jnp.concatenate([dst, n + pad_idx % (npad - n)])
    src32 = src_p.reshape(N_TILES, ch, K)
    dst32 = dst_p.reshape(N_TILES, ch, K)
    ch5 = e_pad // (N_SUB * K)
    src16 = src_p.reshape(N_SUB, ch5, K)
    dst16 = dst_p.reshape(N_SUB, ch5, K)

    zvec = jnp.zeros((npad,), jnp.float32)
    zrows = jnp.zeros((npad, d_hid), jnp.float32)
    x_pad = jnp.pad(x, ((0, npad - n), (0, 0)))

    deg = _make_deg_kernel(npad, ch)(dst32, zvec)

    rb = 512
    grid = (npad // rb,)
    g, dinv2d = pl.pallas_call(
        _g_body,
        grid=grid,
        in_specs=[
            pl.BlockSpec((rb, d_in), lambda i: (i, 0)),
            pl.BlockSpec((d_in, d_hid), lambda i: (0, 0)),
            pl.BlockSpec((rb, 2), lambda i: (i, 0)),
        ],
        out_specs=[
            pl.BlockSpec((rb, d_hid), lambda i: (i, 0)),
            pl.BlockSpec((rb, 1), lambda i: (i, 0)),
        ],
        out_shape=[
            jax.ShapeDtypeStruct((npad, d_hid), jnp.float32),
            jax.ShapeDtypeStruct((npad, 1), jnp.float32),
        ],
    )(x_pad, W1, deg.T)

    k3 = 64
    ch3 = e_pad // (N_TILES * k3)
    src_pk = src_p[0::2] | (src_p[1::2] << 16)
    dst_pk = dst_p[0::2] | (dst_p[1::2] << 16)
    acc = _make_row_scatter_kernel(npad, d_hid, ch3, k3, 16)(
        g, src_pk.reshape(N_TILES, ch3, k3 // 2),
        dst_pk.reshape(N_TILES, ch3, k3 // 2), zrows)

    rb4 = rb
    z2d = pl.pallas_call(
        _l4_body,
        grid=grid,
        in_specs=[
            pl.BlockSpec((2, rb4, d_hid), lambda i: (0, i, 0)),
            pl.BlockSpec((rb4, d_hid), lambda i: (i, 0)),
            pl.BlockSpec((rb4, 1), lambda i: (i, 0)),
            pl.BlockSpec((d_hid, 1), lambda i: (0, 0)),
            pl.BlockSpec((1, d_hid), lambda i: (0, 0)),
        ],
        out_specs=pl.BlockSpec((rb4, 1), lambda i: (i, 0)),
        out_shape=jax.ShapeDtypeStruct((npad, 1), jnp.float32),
    )(acc, g, dinv2d, W2, b1.reshape(1, d_hid))

    b2_b = jnp.broadcast_to(b2, (LANES,)).astype(jnp.float32)
    out5 = _make_scalar_kernel(npad, ch5)(
        z2d.reshape(npad // 128, 128), dinv2d.reshape(npad), b2_b, src16, dst16, zvec)

    return out5[0, :n]


# speedup vs baseline: 1.2599x; 1.2599x over previous
import functools

import jax
import jax.numpy as jnp
from jax import lax
from jax.experimental import pallas as pl
from jax.experimental.pallas import tpu as pltpu
from jax.experimental.pallas import tpu_sc as plsc

N_TILES = 32
N_SUB = 16
K = 128
LANES = 16
NBUF = 2


def _mesh():
    return plsc.VectorSubcoreMesh(core_axis_name="c", subcore_axis_name="s")


def _sc_params():
    return pltpu.CompilerParams(needs_layout_passes=False)


def _make_deg_kernel(npad, ch):
    sl = npad // N_SUB
    assert ch % 8 == 0

    @functools.partial(
        pl.kernel,
        out_type=jax.ShapeDtypeStruct((2, npad), jnp.float32),
        mesh=_mesh(),
        compiler_params=_sc_params(),
        scratch_types=[
            pltpu.VMEM_SHARED((npad,), jnp.float32),
            pltpu.VMEM((ch, K), jnp.int32),
            pltpu.VMEM((K,), jnp.float32),
            pltpu.SemaphoreType.DMA,
        ],
    )
    def deg_kernel(dst_hbm, zvec_hbm, out_hbm, deg_sp, dst_v, ones_v, sem):
        c = lax.axis_index("c")
        s = lax.axis_index("s")
        w = c * N_SUB + s
        pltpu.sync_copy(zvec_hbm.at[pl.ds(s * sl, sl)], deg_sp.at[pl.ds(s * sl, sl)])
        pltpu.sync_copy(dst_hbm.at[w], dst_v)
        for k in range(K // LANES):
            ones_v[pl.ds(k * LANES, LANES)] = jnp.full((LANES,), 1.0, jnp.float32)
        plsc.subcore_barrier()

        def body(i0, carry):
            descs = []
            for b in range(8):
                i = i0 * 8 + b
                descs.append(
                    pltpu.async_copy(ones_v, deg_sp.at[dst_v.at[i]], sem, add=True))
            for d in descs:
                d.wait()
            return carry

        lax.fori_loop(0, ch // 8, body, 0)
        plsc.subcore_barrier()
        pltpu.sync_copy(deg_sp.at[pl.ds(s * sl, sl)], out_hbm.at[c, pl.ds(s * sl, sl)])

    return deg_kernel


def _make_row_scatter_kernel(npad, dh, ch):
    sl = npad // N_SUB

    @functools.partial(
        pl.kernel,
        out_type=jax.ShapeDtypeStruct((2, npad, dh), jnp.float32),
        mesh=_mesh(),
        compiler_params=_sc_params(),
        scratch_types=[
            pltpu.VMEM_SHARED((npad, dh), jnp.float32),
            pltpu.VMEM((ch, K), jnp.int32),
            pltpu.VMEM((ch, K), jnp.int32),
            pltpu.VMEM((K, dh), jnp.float32),
            pltpu.SemaphoreType.DMA,
        ],
    )
    def scat_kernel(g_hbm, src_hbm, dst_hbm, zrows_hbm, out_hbm,
                    acc_sp, src_v, dst_v, rows_a, sem_a):
        c = lax.axis_index("c")
        s = lax.axis_index("s")
        w = c * N_SUB + s
        pltpu.sync_copy(zrows_hbm.at[pl.ds(s * sl, sl)], acc_sp.at[pl.ds(s * sl, sl)])
        pltpu.sync_copy(src_hbm.at[w], src_v)
        pltpu.sync_copy(dst_hbm.at[w], dst_v)
        plsc.subcore_barrier()

        def body2(i, carry):
            pltpu.async_copy(g_hbm.at[src_v.at[i]], rows_a, sem_a).wait()
            pltpu.sync_copy(rows_a, acc_sp.at[dst_v.at[i]], add=True)
            return carry

        lax.fori_loop(0, ch, body2, 0)
        plsc.subcore_barrier()
        pltpu.sync_copy(acc_sp.at[pl.ds(s * sl, sl)], out_hbm.at[c, pl.ds(s * sl, sl)])

    return scat_kernel


def _make_scalar_kernel(npad, ch):
    sl = npad // N_SUB
    assert ch % 2 == 0

    @functools.partial(
        pl.kernel,
        out_type=jax.ShapeDtypeStruct((2, npad), jnp.float32),
        mesh=_mesh(),
        compiler_params=_sc_params(),
        scratch_types=[
            pltpu.VMEM_SHARED((npad,), jnp.float32),
            pltpu.VMEM((ch, K), jnp.int32),
            pltpu.VMEM((ch, K), jnp.int32),
            pltpu.VMEM((npad // 128, 128), jnp.float32),
            pltpu.VMEM((K,), jnp.float32),
            pltpu.VMEM((K,), jnp.float32),
            pltpu.VMEM((sl,), jnp.float32),
            pltpu.VMEM((sl,), jnp.float32),
            pltpu.VMEM((LANES,), jnp.float32),
            pltpu.SemaphoreType.DMA,
            pltpu.SemaphoreType.DMA,
        ],
    )
    def l2agg_kernel(z_hbm, dinv_hbm, b2_hbm, src_hbm, dst_hbm, zvec_hbm, out_hbm,
                     acc_sp, src_v, dst_v, z_v, upd_a, upd_b, acc_v, dinv_v, b2_v,
                     sem_a, sem_b):
        c = lax.axis_index("c")
        s = lax.axis_index("s")
        pltpu.sync_copy(zvec_hbm.at[pl.ds(s * sl, sl)], acc_sp.at[pl.ds(s * sl, sl)])
        pltpu.sync_copy(src_hbm.at[s], src_v)
        pltpu.sync_copy(dst_hbm.at[s], dst_v)
        pltpu.sync_copy(z_hbm, z_v)
        pltpu.sync_copy(dinv_hbm.at[pl.ds(s * sl, sl)], dinv_v)
        pltpu.sync_copy(b2_hbm, b2_v)
        plsc.subcore_barrier()

        def gather_chunk(i, upd_v):
            for k in range(K // LANES):
                s16 = src_v[i, pl.ds(k * LANES, LANES)]
                r16 = lax.shift_right_logical(s16, 7)
                c16 = lax.bitwise_and(s16, 127)
                upd_v[pl.ds(k * LANES, LANES)] = plsc.load_gather(z_v, [r16, c16])

        def body(i0, carry):
            i = i0 * 2
            gather_chunk(i, upd_a)
            da = pltpu.async_copy(upd_a, acc_sp.at[dst_v.at[i]], sem_a, add=True)
            gather_chunk(i + 1, upd_b)
            db = pltpu.async_copy(upd_b, acc_sp.at[dst_v.at[i + 1]], sem_b, add=True)
            da.wait()
            db.wait()
            return carry

        lax.fori_loop(0, ch // 2, body, 0)
        plsc.subcore_barrier()
        pltpu.sync_copy(acc_sp.at[pl.ds(s * sl, sl)], acc_v)
        b2 = b2_v[...]
        rows_per_sub = sl // 128
        for t in range(sl // LANES):
            a16 = acc_v[pl.ds(t * LANES, LANES)]
            z16 = z_v[s * rows_per_sub + t // 8, pl.ds((t % 8) * LANES, LANES)]
            d16 = dinv_v[pl.ds(t * LANES, LANES)]
            acc_v[pl.ds(t * LANES, LANES)] = b2 + d16 * (a16 + z16)
        pltpu.sync_copy(acc_v, out_hbm.at[c, pl.ds(s * sl, sl)])

    return l2agg_kernel


def _g_body(x_ref, w1_ref, degt_ref, g_ref, dinv_ref):
    d = lax.rsqrt(degt_ref[:, 0:1] + degt_ref[:, 1:2] + 1.0)
    h = jnp.dot(x_ref[...], w1_ref[...], preferred_element_type=jnp.float32)
    g_ref[...] = d * h
    dinv_ref[...] = d


def _l4_body(acc_ref, g_ref, dinv_ref, w2_ref, b1_ref, z_ref):
    d = dinv_ref[...]
    h1 = jnp.maximum(d * (acc_ref[0] + acc_ref[1] + g_ref[...]) + b1_ref[...], 0.0)
    z_ref[...] = d * jnp.dot(h1, w2_ref[...], preferred_element_type=jnp.float32)


def kernel(x, edge_index, W1, b1, W2, b2):
    n, d_in = x.shape
    d_hid = W1.shape[1]
    dh = d_hid // 2
    ei = edge_index.astype(jnp.int32)
    src, dst = ei[0], ei[1]
    e = src.shape[0]

    npad = ((n + 16 * 40 - 1) // (16 * 40)) * (16 * 40)
    ch = ((e + N_TILES * K - 1) // (N_TILES * K) + 7) // 8 * 8
    e_pad = N_TILES * ch * K
    n_extra = e_pad - e
    pad_idx = jnp.arange(n_extra, dtype=jnp.int32)
    src_p = jnp.concatenate([src, pad_idx % n])
    dst_p = jnp.concatenate([dst, n + pad_idx % (npad - n)])
    src32 = src_p.reshape(N_TILES, ch, K)
    dst32 = dst_p.reshape(N_TILES, ch, K)
    ch5 = e_pad // (N_SUB * K)
    src16 = src_p.reshape(N_SUB, ch5, K)
    dst16 = dst_p.reshape(N_SUB, ch5, K)

    zvec = jnp.zeros((npad,), jnp.float32)
    zrows = jnp.zeros((npad, d_hid), jnp.float32)
    x_pad = jnp.pad(x, ((0, npad - n), (0, 0)))

    deg = _make_deg_kernel(npad, ch)(dst32, zvec)

    rb = 512
    grid = (npad // rb,)
    g, dinv2d = pl.pallas_call(
        _g_body,
        grid=grid,
        in_specs=[
            pl.BlockSpec((rb, d_in), lambda i: (i, 0)),
            pl.BlockSpec((d_in, d_hid), lambda i: (0, 0)),
            pl.BlockSpec((rb, 2), lambda i: (i, 0)),
        ],
        out_specs=[
            pl.BlockSpec((rb, d_hid), lambda i: (i, 0)),
            pl.BlockSpec((rb, 1), lambda i: (i, 0)),
        ],
        out_shape=[
            jax.ShapeDtypeStruct((npad, d_hid), jnp.float32),
            jax.ShapeDtypeStruct((npad, 1), jnp.float32),
        ],
    )(x_pad, W1, deg.T)

    acc = _make_row_scatter_kernel(npad, d_hid, ch)(g, src32, dst32, zrows)

    rb4 = rb
    z2d = pl.pallas_call(
        _l4_body,
        grid=grid,
        in_specs=[
            pl.BlockSpec((2, rb4, d_hid), lambda i: (0, i, 0)),
            pl.BlockSpec((rb4, d_hid), lambda i: (i, 0)),
            pl.BlockSpec((rb4, 1), lambda i: (i, 0)),
            pl.BlockSpec((d_hid, 1), lambda i: (0, 0)),
            pl.BlockSpec((1, d_hid), lambda i: (0, 0)),
        ],
        out_specs=pl.BlockSpec((rb4, 1), lambda i: (i, 0)),
        out_shape=jax.ShapeDtypeStruct((npad, 1), jnp.float32),
    )(acc, g, dinv2d, W2, b1.reshape(1, d_hid))

    b2_b = jnp.broadcast_to(b2, (LANES,)).astype(jnp.float32)
    out5 = _make_scalar_kernel(npad, ch5)(
        z2d.reshape(npad // 128, 128), dinv2d.reshape(npad), b2_b, src16, dst16, zvec)

    return out5[0, :n]
